# Initial kernel scaffold; baseline (speedup 1.0000x reference)
import jax
import jax.numpy as jnp
from jax.experimental import pallas as pl


def kernel(center, context, negatives, apply_dropout, center_table, context_table):
    raise NotImplementedError("write your pallas kernel here")


# speedup vs baseline: 5.3979x; 5.3979x over previous
import functools

import jax
import jax.numpy as jnp
from jax import lax
from jax.experimental import pallas as pl
from jax.experimental.pallas import tpu as pltpu
from jax.experimental.pallas import tpu_sc as plsc

NC = 2
NS = 16
L = 16
NW = NC * NS

CHUNK = 64
LS = 0.1


def _softplus(t):
    m = jnp.maximum(t, 0.0)
    y = jnp.exp(-jnp.abs(t))
    w = y / (y + 2.0)
    w2 = w * w
    p = w * (2.0 + w2 * (2.0 / 3.0 + w2 * (0.4 + w2 * (2.0 / 7.0 + w2 * (2.0 / 9.0)))))
    return m + p


@functools.partial(jax.jit, static_argnames=("B", "K", "D"))
def _skipgram_sc(center, context, neg2d, ctab, xtab, *, B, K, D):
    per_w = B // NW
    n_chunks = per_w // CHUNK
    groups = CHUNK // L
    nrow_per_w = per_w * K // 128
    nrow_per_chunk = CHUNK * K // 128

    mesh = plsc.VectorSubcoreMesh(core_axis_name="c", subcore_axis_name="s")

    @functools.partial(
        pl.kernel,
        mesh=mesh,
        out_type=jax.ShapeDtypeStruct((B,), jnp.float32),
        compiler_params=pltpu.CompilerParams(
            needs_layout_passes=False, use_tc_tiling_on_sc=False),
        scratch_types=[
            pltpu.VMEM((per_w,), jnp.int32),
            pltpu.VMEM((per_w,), jnp.int32),
            pltpu.VMEM((nrow_per_w, 128), jnp.int32),
            pltpu.VMEM((CHUNK, D), jnp.float32),
            pltpu.VMEM((CHUNK, D), jnp.float32),
            pltpu.VMEM((CHUNK * K, D), jnp.float32),
            pltpu.VMEM((CHUNK,), jnp.float32),
            pltpu.SemaphoreType.DMA,
        ],
    )
    def sc_kernel(center_hbm, context_hbm, neg_hbm, ctab_hbm, xtab_hbm,
                  out_hbm, cidx_v, xidx_v, nidx_v, cemb_v, xemb_v, nemb_v,
                  out_v, sem):
        wid = lax.axis_index("s") * NC + lax.axis_index("c")
        wbase = wid * per_w

        pltpu.sync_copy(center_hbm.at[pl.ds(wbase, per_w)], cidx_v)
        pltpu.sync_copy(context_hbm.at[pl.ds(wbase, per_w)], xidx_v)
        pltpu.sync_copy(neg_hbm.at[pl.ds(wid * nrow_per_w, nrow_per_w)], nidx_v)

        def chunk_body(c, _):
            base = wbase + c * CHUNK
            copies = [
                pltpu.async_copy(
                    ctab_hbm.at[cidx_v.at[pl.ds(c * CHUNK, CHUNK)]], cemb_v, sem),
                pltpu.async_copy(
                    xtab_hbm.at[xidx_v.at[pl.ds(c * CHUNK, CHUNK)]], xemb_v, sem),
            ]
            for i in range(nrow_per_chunk):
                copies.append(pltpu.async_copy(
                    xtab_hbm.at[nidx_v.at[c * nrow_per_chunk + i]],
                    nemb_v.at[pl.ds(i * 128, 128)], sem))
            for cp in copies:
                cp.wait()

            zero = jnp.zeros((L,), jnp.float32)
            lane = lax.iota(jnp.int32, L)

            def g_body(g, _):
                def l_body(l, scores):
                    r = g * L + l
                    mask = lane == l
                    cvecs = [cemb_v[r, pl.ds(j * L, L)] for j in range(D // L)]
                    pv = cvecs[0] * xemb_v[r, pl.ds(0, L)]
                    for j in range(1, D // L):
                        pv = pv + cvecs[j] * xemb_v[r, pl.ds(j * L, L)]
                    new = [jnp.where(mask, jnp.sum(pv), scores[0])]
                    for k in range(K):
                        nr = r * K + k
                        nv = cvecs[0] * nemb_v[nr, pl.ds(0, L)]
                        for j in range(1, D // L):
                            nv = nv + cvecs[j] * nemb_v[nr, pl.ds(j * L, L)]
                        new.append(jnp.where(mask, jnp.sum(nv), scores[k + 1]))
                    return tuple(new)

                scores = lax.fori_loop(0, L, l_body, (zero,) * (K + 1))
                s = scores[0]
                loss = _softplus(-s) + LS * s
                for n in scores[1:]:
                    loss = loss + _softplus(-n) + (1.0 - LS) * n
                out_v[pl.ds(g * L, L)] = loss
                return ()

            lax.fori_loop(0, groups, g_body, ())

            pltpu.sync_copy(out_v, out_hbm.at[pl.ds(base, CHUNK)])
            return ()

        lax.fori_loop(0, n_chunks, chunk_body, ())

    return sc_kernel(center, context, neg2d, ctab, xtab)


def kernel(center, context, negatives, apply_dropout, center_table, context_table):
    del apply_dropout
    B, = center.shape
    K = negatives.shape[1]
    D = center_table.shape[1]
    neg2d = negatives.astype(jnp.int32).reshape(B * K // 128, 128)
    return _skipgram_sc(center.astype(jnp.int32), context.astype(jnp.int32),
                        neg2d, center_table, context_table, B=B, K=K, D=D)
